# Initial kernel scaffold; baseline (speedup 1.0000x reference)
import jax
import jax.numpy as jnp
from jax.experimental import pallas as pl


def kernel(x, x_lengths, weight, leaf_ancestor_matrix, inf_adjacency_matrix):
    raise NotImplementedError("write your pallas kernel here")


# speedup vs baseline: 38.3204x; 38.3204x over previous
import functools

import jax
import jax.numpy as jnp
from jax import lax
from jax.experimental import pallas as pl
from jax.experimental.pallas import tpu as pltpu
from jax.experimental.pallas import tpu_sc as plsc

_NUM_ITEMS = 256
_NUM_NESTS = 16
_NUM_NODES = 1 + _NUM_NESTS + _NUM_ITEMS
_PAD = _NUM_ITEMS
_B = 512
_S = 20
_NC = 2
_NS = 16
_NW = _NC * _NS
_ROWS = _B // _NW
_WPAD = 288


def _ln(x):
    bits = lax.bitcast_convert_type(x, jnp.int32)
    y = bits.astype(jnp.float32) * jnp.float32(8.2629582e-08) - jnp.float32(87.98997156)
    for _ in range(3):
        y = y + x * jnp.exp(-y) - jnp.float32(1.0)
    return y


@functools.cache
def _build_sc_kernel():
    mesh = plsc.VectorSubcoreMesh(core_axis_name="c", subcore_axis_name="s")
    return pl.kernel(
        _nested_logit_sc,
        mesh=mesh,
        compiler_params=pltpu.CompilerParams(needs_layout_passes=False),
        out_type=jax.ShapeDtypeStruct((_NW, _S, _ROWS), jnp.float32),
        scratch_types=[
            pltpu.VMEM((_S, _ROWS), jnp.int32),
            pltpu.VMEM((_ROWS,), jnp.int32),
            pltpu.VMEM((_WPAD,), jnp.float32),
            pltpu.VMEM((_S, _ROWS), jnp.float32),
        ],
    )


def _nested_logit_sc(x_hbm, len_hbm, w_hbm, out_hbm, x_v, len_v, w_v, out_v):
    wid = lax.axis_index("s") * _NC + lax.axis_index("c")
    pltpu.sync_copy(x_hbm.at[wid], x_v)
    pltpu.sync_copy(len_hbm.at[wid], len_v)
    pltpu.sync_copy(w_hbm, w_v)

    xs = [x_v[s, :] for s in range(_S)]
    gs = [jnp.right_shift(xs[s], 4) for s in range(_S)]
    u = [plsc.load_gather(w_v, [xs[s] + 17]) for s in range(_S)]
    nu = [plsc.load_gather(w_v, [gs[s] + 1]) for s in range(_S)]

    fi = [None] * _S
    fn = [None] * _S
    for s in range(1, _S):
        ai = xs[0] != xs[s]
        an = gs[0] != gs[s]
        for t in range(1, s):
            ai = ai & (xs[t] != xs[s])
            an = an & (gs[t] != gs[s])
        fi[s] = ai
        fn[s] = an

    c = u[0]
    for s in range(1, _S):
        c = jnp.maximum(c, u[s])
    e = [jnp.exp(u[s] - c) for s in range(_S)]
    fe = [e[0]] + [jnp.where(fi[s], e[s], jnp.float32(0.0)) for s in range(1, _S)]
    zn = [None] * _S
    for s in range(_S):
        acc = fe[0] if s == 0 else jnp.where(gs[0] == gs[s], fe[0], jnp.float32(0.0))
        for t in range(1, _S):
            acc = acc + jnp.where(gs[t] == gs[s], fe[t], jnp.float32(0.0))
        zn[s] = c + _ln(acc)

    cn = nu[0]
    for s in range(1, _S):
        cn = jnp.maximum(cn, nu[s])
    sr = jnp.exp(nu[0] - cn)
    for s in range(1, _S):
        sr = sr + jnp.where(fn[s], jnp.exp(nu[s] - cn), jnp.float32(0.0))
    zr = cn + _ln(sr)

    neg_inf = jnp.float32(-jnp.inf)
    lens = len_v[:]
    for s in range(_S):
        val = u[s] - zn[s] + nu[s] - zr
        out_v[s, :] = jnp.where(lens > s, val, neg_inf)

    pltpu.sync_copy(out_v, out_hbm.at[wid])


def kernel(x, x_lengths, weight, leaf_ancestor_matrix, inf_adjacency_matrix):
    del leaf_ancestor_matrix, inf_adjacency_matrix
    Bn, Sn = x.shape
    w = weight[:, 0].at[_PAD].set(0.0)
    w_pad = jnp.zeros((_WPAD,), jnp.float32).at[: _NUM_NODES + 1].set(w)
    x_k = x.reshape(_NW, _ROWS, Sn).transpose(0, 2, 1)
    len_k = x_lengths.reshape(_NW, _ROWS)
    out_k = _build_sc_kernel()(x_k, len_k, w_pad)
    return out_k.transpose(0, 2, 1).reshape(Bn, Sn, 1)


# speedup vs baseline: 41.4767x; 1.0824x over previous
import functools

import jax
import jax.numpy as jnp
from jax import lax
from jax.experimental import pallas as pl
from jax.experimental.pallas import tpu as pltpu
from jax.experimental.pallas import tpu_sc as plsc

_NUM_ITEMS = 256
_NUM_NESTS = 16
_NUM_NODES = 1 + _NUM_NESTS + _NUM_ITEMS
_B = 512
_S = 20
_NC = 2
_NS = 16
_NW = _NC * _NS
_ROWS = _B // _NW
_L = 16


def _ln(x):
    bits = lax.bitcast_convert_type(x, jnp.int32)
    y = bits.astype(jnp.float32) * jnp.float32(8.2629582e-08) - jnp.float32(87.98997156)
    for _ in range(3):
        y = y + x * jnp.exp(-y) - jnp.float32(1.0)
    return y


def _nested_logit_body(x_hbm, len_hbm, w_hbm, out_hbm,
                       x_v, len_v, w_v, out_v, mark_i, mark_n, acc, sems):
    wid = lax.axis_index("s") * _NC + lax.axis_index("c")
    base = wid * _ROWS
    cp_x = pltpu.async_copy(x_hbm.at[pl.ds(base, _ROWS), :], x_v, sems.at[0])
    cp_l = pltpu.async_copy(len_hbm.at[pl.ds(base, _ROWS)], len_v, sems.at[1])
    cp_w = pltpu.async_copy(w_hbm, w_v, sems.at[2])
    cp_x.wait()
    cp_l.wait()
    cp_w.wait()

    lanes = lax.iota(jnp.int32, _L)

    xs = [plsc.load_gather(x_v, [lanes, jnp.full((_L,), s, jnp.int32)])
          for s in range(_S)]
    gs = [jnp.right_shift(xs[s], 4) for s in range(_S)]
    u = [plsc.load_gather(w_v, [xs[s] + 17]) for s in range(_S)]
    u = [jnp.where(xs[s] == _NUM_ITEMS - 17, jnp.float32(0.0), u[s])
         for s in range(_S)]
    nu = [plsc.load_gather(w_v, [gs[s] + 1]) for s in range(_S)]

    for s in range(_S - 1, -1, -1):
        sv = jnp.full((_L,), s, jnp.int32)
        plsc.store_scatter(mark_i, [xs[s], lanes], sv)
        plsc.store_scatter(mark_n, [gs[s], lanes], sv)
    fi = [plsc.load_gather(mark_i, [xs[s], lanes]) == s for s in range(_S)]
    fn = [plsc.load_gather(mark_n, [gs[s], lanes]) == s for s in range(_S)]

    c = u[0]
    for s in range(1, _S):
        c = jnp.maximum(c, u[s])
    zero = jnp.zeros((_L,), jnp.float32)
    for g in range(_NUM_NESTS):
        acc[g, :] = zero
    for s in range(_S):
        e = jnp.where(fi[s], jnp.exp(u[s] - c), jnp.float32(0.0))
        plsc.addupdate_scatter(acc, [gs[s], lanes], e)
    zn = [c + _ln(plsc.load_gather(acc, [gs[s], lanes])) for s in range(_S)]

    cn = nu[0]
    for s in range(1, _S):
        cn = jnp.maximum(cn, nu[s])
    sr = jnp.exp(nu[0] - cn)
    for s in range(1, _S):
        sr = sr + jnp.where(fn[s], jnp.exp(nu[s] - cn), jnp.float32(0.0))
    zr = cn + _ln(sr)

    neg_inf = jnp.float32(-jnp.inf)
    lens = len_v[:]
    for s in range(_S):
        val = u[s] - zn[s] + nu[s] - zr
        val = jnp.where(lens > s, val, neg_inf)
        plsc.store_scatter(out_v, [lanes, jnp.full((_L,), s, jnp.int32)], val)

    pltpu.sync_copy(out_v, out_hbm.at[pl.ds(base, _ROWS), :])


@functools.cache
def _build_sc_kernel():
    mesh = plsc.VectorSubcoreMesh(core_axis_name="c", subcore_axis_name="s")
    return pl.kernel(
        _nested_logit_body,
        mesh=mesh,
        compiler_params=pltpu.CompilerParams(needs_layout_passes=False),
        out_type=jax.ShapeDtypeStruct((_B, _S), jnp.float32),
        scratch_types=[
            pltpu.VMEM((_ROWS, _S), jnp.int32),
            pltpu.VMEM((_ROWS,), jnp.int32),
            pltpu.VMEM((_NUM_NODES + 1,), jnp.float32),
            pltpu.VMEM((_ROWS, _S), jnp.float32),
            pltpu.VMEM((_NUM_ITEMS, _L), jnp.int32),
            pltpu.VMEM((_NUM_NESTS, _L), jnp.int32),
            pltpu.VMEM((_NUM_NESTS, _L), jnp.float32),
            pltpu.SemaphoreType.DMA((3,)),
        ],
    )


def kernel(x, x_lengths, weight, leaf_ancestor_matrix, inf_adjacency_matrix):
    del leaf_ancestor_matrix, inf_adjacency_matrix
    Bn, Sn = x.shape
    out = _build_sc_kernel()(x, x_lengths, weight.reshape(_NUM_NODES + 1))
    return out.reshape(Bn, Sn, 1)


# speedup vs baseline: 42.1917x; 1.0172x over previous
import functools

import jax
import jax.numpy as jnp
from jax import lax
from jax.experimental import pallas as pl
from jax.experimental.pallas import tpu as pltpu
from jax.experimental.pallas import tpu_sc as plsc

_NUM_ITEMS = 256
_NUM_NESTS = 16
_NUM_NODES = 1 + _NUM_NESTS + _NUM_ITEMS
_B = 512
_S = 20
_NC = 2
_NS = 16
_NW = _NC * _NS
_ROWS = _B // _NW
_L = 16


def _ln(x):
    bits = lax.bitcast_convert_type(x, jnp.int32)
    y = bits.astype(jnp.float32) * jnp.float32(8.2629582e-08) - jnp.float32(87.98997156)
    for _ in range(2):
        y = y + x * jnp.exp(-y) - jnp.float32(1.0)
    return y


def _nested_logit_body(x_hbm, len_hbm, w_hbm, out_hbm,
                       x_v, len_v, w_v, out_v, mark_i, mark_n, acc, sems):
    wid = lax.axis_index("s") * _NC + lax.axis_index("c")
    base = wid * _ROWS
    cp_x = pltpu.async_copy(x_hbm.at[pl.ds(base, _ROWS), :], x_v, sems.at[0])
    cp_l = pltpu.async_copy(len_hbm.at[pl.ds(base, _ROWS)], len_v, sems.at[1])
    cp_w = pltpu.async_copy(w_hbm, w_v, sems.at[2])
    cp_x.wait()
    cp_l.wait()
    cp_w.wait()

    lanes = lax.iota(jnp.int32, _L)

    seg = w_v[pl.ds(_NUM_ITEMS, _L)]
    w_v[pl.ds(_NUM_ITEMS, _L)] = jnp.where(lanes == 0, jnp.float32(0.0), seg)

    def _pos(s):
        xs = plsc.load_gather(x_v, [lanes, jnp.broadcast_to(s, (_L,))])
        gs = jnp.right_shift(xs, 4)
        return xs, gs

    neg_inf = jnp.float32(-jnp.inf)

    def _phase_a(j, carry):
        c, cn = carry
        s = _S - 1 - j
        xs, gs = _pos(s)
        sv = jnp.broadcast_to(s, (_L,))
        plsc.store_scatter(mark_i, [xs, lanes], sv)
        plsc.store_scatter(mark_n, [gs, lanes], sv)
        u = plsc.load_gather(w_v, [xs + 17])
        nu = plsc.load_gather(w_v, [gs + 1])
        return jnp.maximum(c, u), jnp.maximum(cn, nu)

    c, cn = lax.fori_loop(
        0, _S, _phase_a,
        (jnp.full((_L,), neg_inf), jnp.full((_L,), neg_inf)))

    zero = jnp.zeros((_L,), jnp.float32)

    def _init_acc(g, carry):
        acc[g, :] = zero
        return carry

    lax.fori_loop(0, _NUM_NESTS, _init_acc, 0)

    def _phase_b(s, sr):
        xs, gs = _pos(s)
        u = plsc.load_gather(w_v, [xs + 17])
        nu = plsc.load_gather(w_v, [gs + 1])
        fi = plsc.load_gather(mark_i, [xs, lanes]) == s
        fn = plsc.load_gather(mark_n, [gs, lanes]) == s
        e = jnp.where(fi, jnp.exp(u - c), jnp.float32(0.0))
        plsc.addupdate_scatter(acc, [gs, lanes], e)
        return sr + jnp.where(fn, jnp.exp(nu - cn), jnp.float32(0.0))

    sr = lax.fori_loop(0, _S, _phase_b, zero)

    ccn = c + cn
    lens = len_v[:]

    def _phase_c(s, carry):
        xs, gs = _pos(s)
        u = plsc.load_gather(w_v, [xs + 17])
        nu = plsc.load_gather(w_v, [gs + 1])
        a = plsc.load_gather(acc, [gs, lanes])
        val = u + nu - ccn - _ln(a * sr)
        val = jnp.where(lens > s, val, neg_inf)
        plsc.store_scatter(out_v, [lanes, jnp.broadcast_to(s, (_L,))], val)
        return carry

    lax.fori_loop(0, _S, _phase_c, 0)

    pltpu.sync_copy(out_v, out_hbm.at[pl.ds(base, _ROWS), :])


@functools.cache
def _build_sc_kernel():
    mesh = plsc.VectorSubcoreMesh(core_axis_name="c", subcore_axis_name="s")
    return pl.kernel(
        _nested_logit_body,
        mesh=mesh,
        compiler_params=pltpu.CompilerParams(needs_layout_passes=False),
        out_type=jax.ShapeDtypeStruct((_B, _S), jnp.float32),
        scratch_types=[
            pltpu.VMEM((_ROWS, _S), jnp.int32),
            pltpu.VMEM((_ROWS,), jnp.int32),
            pltpu.VMEM((_NUM_NODES + 1,), jnp.float32),
            pltpu.VMEM((_ROWS, _S), jnp.float32),
            pltpu.VMEM((_NUM_ITEMS, _L), jnp.int32),
            pltpu.VMEM((_NUM_NESTS, _L), jnp.int32),
            pltpu.VMEM((_NUM_NESTS, _L), jnp.float32),
            pltpu.SemaphoreType.DMA((3,)),
        ],
    )


def kernel(x, x_lengths, weight, leaf_ancestor_matrix, inf_adjacency_matrix):
    del leaf_ancestor_matrix, inf_adjacency_matrix
    Bn, Sn = x.shape
    out = _build_sc_kernel()(x, x_lengths, weight.reshape(_NUM_NODES + 1))
    return out.reshape(Bn, Sn, 1)


# speedup vs baseline: 43.3908x; 1.0284x over previous
import functools

import jax
import jax.numpy as jnp
from jax import lax
from jax.experimental import pallas as pl
from jax.experimental.pallas import tpu as pltpu
from jax.experimental.pallas import tpu_sc as plsc

_NUM_ITEMS = 256
_NUM_NESTS = 16
_NUM_NODES = 1 + _NUM_NESTS + _NUM_ITEMS
_B = 512
_S = 20
_NC = 2
_NS = 16
_NW = _NC * _NS
_ROWS = _B // _NW
_L = 16


def _ln(x):
    bits = lax.bitcast_convert_type(x, jnp.int32)
    y = bits.astype(jnp.float32) * jnp.float32(8.2629582e-08) - jnp.float32(87.98997156)
    for _ in range(2):
        y = y + x * jnp.exp(-y) - jnp.float32(1.0)
    return y


def _nested_logit_body(x_hbm, len_hbm, w_hbm, out_hbm,
                       x_v, len_v, w_v, out_v, mark_i, mark_n, acc, sems):
    wid = lax.axis_index("s") * _NC + lax.axis_index("c")
    base = wid * _ROWS
    col = (wid % 8) * _ROWS
    cp_x = pltpu.async_copy(
        x_hbm.at[:, pl.ds(pl.multiple_of((wid // 8) * 128, 128), 128)],
        x_v, sems.at[0])
    cp_l = pltpu.async_copy(len_hbm.at[pl.ds(base, _ROWS)], len_v, sems.at[1])
    cp_w = pltpu.async_copy(w_hbm, w_v, sems.at[2])
    cp_x.wait()
    cp_l.wait()
    cp_w.wait()

    lanes = lax.iota(jnp.int32, _L)

    seg = w_v[pl.ds(_NUM_ITEMS, _L)]
    w_v[pl.ds(_NUM_ITEMS, _L)] = jnp.where(lanes == 0, jnp.float32(0.0), seg)

    def _pos(s):
        xs = plsc.load_gather(x_v, [jnp.broadcast_to(s, (_L,)), col + lanes])
        gs = jnp.right_shift(xs, 4)
        return xs, gs

    neg_inf = jnp.float32(-jnp.inf)

    def _phase_a(j, carry):
        c, cn = carry
        s = _S - 1 - j
        xs, gs = _pos(s)
        sv = jnp.broadcast_to(s, (_L,))
        plsc.store_scatter(mark_i, [xs, lanes], sv)
        plsc.store_scatter(mark_n, [gs, lanes], sv)
        u = plsc.load_gather(w_v, [xs + 17])
        nu = plsc.load_gather(w_v, [gs + 1])
        return jnp.maximum(c, u), jnp.maximum(cn, nu)

    c, cn = lax.fori_loop(
        0, _S, _phase_a,
        (jnp.full((_L,), neg_inf), jnp.full((_L,), neg_inf)))

    zero = jnp.zeros((_L,), jnp.float32)

    def _init_acc(g, carry):
        acc[g, :] = zero
        return carry

    lax.fori_loop(0, _NUM_NESTS, _init_acc, 0)

    def _phase_b(s, sr):
        xs, gs = _pos(s)
        u = plsc.load_gather(w_v, [xs + 17])
        nu = plsc.load_gather(w_v, [gs + 1])
        fi = plsc.load_gather(mark_i, [xs, lanes]) == s
        fn = plsc.load_gather(mark_n, [gs, lanes]) == s
        e = jnp.where(fi, jnp.exp(u - c), jnp.float32(0.0))
        plsc.addupdate_scatter(acc, [gs, lanes], e)
        return sr + jnp.where(fn, jnp.exp(nu - cn), jnp.float32(0.0))

    sr = lax.fori_loop(0, _S, _phase_b, zero)

    ccn = c + cn
    lens = len_v[:]

    def _phase_c(s, carry):
        xs, gs = _pos(s)
        u = plsc.load_gather(w_v, [xs + 17])
        nu = plsc.load_gather(w_v, [gs + 1])
        a = plsc.load_gather(acc, [gs, lanes])
        val = u + nu - ccn - _ln(a * sr)
        val = jnp.where(lens > s, val, neg_inf)
        plsc.store_scatter(out_v, [jnp.broadcast_to(s, (_L,)), lanes], val)
        return carry

    lax.fori_loop(0, _S, _phase_c, 0)

    cps = [pltpu.async_copy(out_v.at[s], out_hbm.at[pl.ds(s * _B + base, _ROWS)],
                            sems.at[3]) for s in range(_S)]
    for cp in cps:
        cp.wait()


@functools.cache
def _build_sc_kernel():
    mesh = plsc.VectorSubcoreMesh(core_axis_name="c", subcore_axis_name="s")
    return pl.kernel(
        _nested_logit_body,
        mesh=mesh,
        compiler_params=pltpu.CompilerParams(needs_layout_passes=False),
        out_type=jax.ShapeDtypeStruct((_B * _S,), jnp.float32),
        scratch_types=[
            pltpu.VMEM((_S, 128), jnp.int32),
            pltpu.VMEM((_ROWS,), jnp.int32),
            pltpu.VMEM((_NUM_NODES + 1,), jnp.float32),
            pltpu.VMEM((_S, _ROWS), jnp.float32),
            pltpu.VMEM((_NUM_ITEMS, _L), jnp.int32),
            pltpu.VMEM((_NUM_NESTS, _L), jnp.int32),
            pltpu.VMEM((_NUM_NESTS, _L), jnp.float32),
            pltpu.SemaphoreType.DMA((4,)),
        ],
    )


def kernel(x, x_lengths, weight, leaf_ancestor_matrix, inf_adjacency_matrix):
    del leaf_ancestor_matrix, inf_adjacency_matrix
    Bn, Sn = x.shape
    out_t = _build_sc_kernel()(x.T, x_lengths, weight.reshape(_NUM_NODES + 1))
    return out_t.reshape(Sn, Bn).T.reshape(Bn, Sn, 1)
